# baseline (speedup 1.0000x reference)
import functools

import jax
import jax.numpy as jnp
from jax.experimental import pallas as pl
from jax.experimental.pallas import tpu as pltpu

_NBUF = 6
_LOOK = 3


def _body(ar_ref, gate_ref, emb_ref, x_ref, o_ref, buf, in_sems, out_sems,
          *, num_tiles, nc):
    g = jnp.tanh(gate_ref[0])

    def in_copy(s, slot):
        b = s // num_tiles
        t = s % num_tiles
        return pltpu.make_async_copy(
            x_ref.at[b, t], buf.at[slot], in_sems.at[slot])

    def out_copy(s, slot):
        b = s // num_tiles
        t = s % num_tiles
        return pltpu.make_async_copy(
            buf.at[slot], o_ref.at[b, t], out_sems.at[slot])

    for k in range(_LOOK):
        in_copy(k, k).start()

    def step(i, carry):
        slot = i % _NBUF
        ahead = i + _LOOK
        slot_a = ahead % _NBUF

        @pl.when(jnp.logical_and(ahead < nc, ahead >= _NBUF))
        def _():
            out_copy(ahead - _NBUF, slot_a).wait()

        @pl.when(ahead < nc)
        def _():
            in_copy(ahead, slot_a).start()

        in_copy(i, slot).wait()

        b = i // num_tiles
        t = i % num_tiles
        h = ar_ref[b, 0]
        w = ar_ref[b, 1]
        ws = jnp.maximum(w, 1)
        e = (t // ws) * num_tiles + t % ws
        sc = jnp.where(t < h * w, g, jnp.float32(0.0))
        row = emb_ref[e, :] * sc
        buf[slot] = buf[slot] + row[None, :]

        out_copy(i, slot).start()
        return carry

    jax.lax.fori_loop(0, nc, step, None)

    for k in range(_NBUF):
        out_copy(0, k).wait()


def kernel(x, ar, embedding, gate):
    bsz, num_tiles, ntok, width = x.shape
    nc = bsz * num_tiles
    emb2 = embedding.reshape(num_tiles * num_tiles, width)

    body = functools.partial(_body, num_tiles=num_tiles, nc=nc)

    grid_spec = pltpu.PrefetchScalarGridSpec(
        num_scalar_prefetch=2,
        grid=(1,),
        in_specs=[
            pl.BlockSpec(emb2.shape, lambda i, *_: (0, 0)),
            pl.BlockSpec(memory_space=pl.ANY),
        ],
        out_specs=pl.BlockSpec(memory_space=pl.ANY),
        scratch_shapes=[
            pltpu.VMEM((_NBUF, ntok, width), jnp.float32),
            pltpu.SemaphoreType.DMA((_NBUF,)),
            pltpu.SemaphoreType.DMA((_NBUF,)),
        ],
    )
    return pl.pallas_call(
        body,
        grid_spec=grid_spec,
        out_shape=jax.ShapeDtypeStruct(x.shape, x.dtype),
        compiler_params=pltpu.CompilerParams(
            dimension_semantics=("arbitrary",),
        ),
    )(ar, gate, emb2, x)


# speedup vs baseline: 3.7490x; 3.7490x over previous
import jax
import jax.numpy as jnp
from jax.experimental import pallas as pl
from jax.experimental.pallas import tpu as pltpu

_BLKN = 201


def _body(ar_ref, gate_ref, x_ref, emb_ref, o_ref, bias_ref, *, num_tiles):
    b = pl.program_id(0)
    g = jnp.tanh(gate_ref[0])
    h = ar_ref[b, 0]
    w = ar_ref[b, 1]
    ws = jnp.maximum(w, 1)
    hw = h * w
    for t in range(num_tiles):
        e = (t // ws) * num_tiles + t % ws
        sc = jnp.where(t < hw, g, jnp.float32(0.0))
        bias_ref[t, :] = emb_ref[e, :] * sc
    o_ref[...] = x_ref[...] + bias_ref[...][None, None, :, :]


def kernel(x, ar, embedding, gate):
    bsz, num_tiles, ntok, width = x.shape
    nblk = (ntok + _BLKN - 1) // _BLKN
    emb2 = embedding.reshape(num_tiles * num_tiles, width)
    xt = jnp.transpose(x, (0, 2, 1, 3))

    import functools
    body = functools.partial(_body, num_tiles=num_tiles)

    def x_map(b, n, ar_ref, gate_ref):
        return (b, n, 0, 0)

    grid_spec = pltpu.PrefetchScalarGridSpec(
        num_scalar_prefetch=2,
        grid=(bsz, nblk),
        in_specs=[
            pl.BlockSpec((1, _BLKN, num_tiles, width), x_map),
            pl.BlockSpec(emb2.shape, lambda b, n, *_: (0, 0)),
        ],
        out_specs=pl.BlockSpec((1, _BLKN, num_tiles, width), x_map),
        scratch_shapes=[
            pltpu.VMEM((num_tiles, width), jnp.float32),
        ],
    )
    out_t = pl.pallas_call(
        body,
        grid_spec=grid_spec,
        out_shape=jax.ShapeDtypeStruct(xt.shape, x.dtype),
        compiler_params=pltpu.CompilerParams(
            dimension_semantics=("parallel", "arbitrary"),
        ),
    )(ar, gate, xt, emb2)
    return jnp.transpose(out_t, (0, 2, 1, 3))


# speedup vs baseline: 3.8034x; 1.0145x over previous
import jax
import jax.numpy as jnp
from jax.experimental import pallas as pl
from jax.experimental.pallas import tpu as pltpu

_BLKN = 402


def _body(ar_ref, gate_ref, x_ref, emb_ref, o_ref, bias_ref, *, num_tiles):
    b = pl.program_id(0)

    @pl.when(pl.program_id(1) == 0)
    def _():
        g = jnp.tanh(gate_ref[0])
        h = ar_ref[b, 0]
        w = ar_ref[b, 1]
        ws = jnp.maximum(w, 1)
        hw = h * w
        for t in range(num_tiles):
            e = (t // ws) * num_tiles + t % ws
            sc = jnp.where(t < hw, g, jnp.float32(0.0))
            bias_ref[t, :] = emb_ref[e, :] * sc

    o_ref[...] = x_ref[...] + bias_ref[...][None, None, :, :]


def kernel(x, ar, embedding, gate):
    bsz, num_tiles, ntok, width = x.shape
    nblk = (ntok + _BLKN - 1) // _BLKN
    emb2 = embedding.reshape(num_tiles * num_tiles, width)
    xt = jnp.transpose(x, (0, 2, 1, 3))

    import functools
    body = functools.partial(_body, num_tiles=num_tiles)

    def x_map(b, n, ar_ref, gate_ref):
        return (b, n, 0, 0)

    grid_spec = pltpu.PrefetchScalarGridSpec(
        num_scalar_prefetch=2,
        grid=(bsz, nblk),
        in_specs=[
            pl.BlockSpec((1, _BLKN, num_tiles, width), x_map),
            pl.BlockSpec(emb2.shape, lambda b, n, *_: (0, 0)),
        ],
        out_specs=pl.BlockSpec((1, _BLKN, num_tiles, width), x_map),
        scratch_shapes=[
            pltpu.VMEM((num_tiles, width), jnp.float32),
        ],
    )
    out_t = pl.pallas_call(
        body,
        grid_spec=grid_spec,
        out_shape=jax.ShapeDtypeStruct(xt.shape, x.dtype),
        compiler_params=pltpu.CompilerParams(
            dimension_semantics=("parallel", "arbitrary"),
        ),
    )(ar, gate, xt, emb2)
    return jnp.transpose(out_t, (0, 2, 1, 3))


# speedup vs baseline: 3.8220x; 1.0049x over previous
import jax
import jax.numpy as jnp
from jax.experimental import pallas as pl
from jax.experimental.pallas import tpu as pltpu

_BLKN = 804


def _body(ar_ref, gate_ref, x_ref, emb_ref, o_ref, bias_ref, *, num_tiles):
    b = pl.program_id(0)

    @pl.when(pl.program_id(1) == 0)
    def _():
        g = jnp.tanh(gate_ref[0])
        h = ar_ref[b, 0]
        w = ar_ref[b, 1]
        ws = jnp.maximum(w, 1)
        hw = h * w
        for t in range(num_tiles):
            e = (t // ws) * num_tiles + t % ws
            sc = jnp.where(t < hw, g, jnp.float32(0.0))
            bias_ref[t, :] = emb_ref[e, :] * sc

    o_ref[...] = x_ref[...] + bias_ref[...][None, None, :, :]


def kernel(x, ar, embedding, gate):
    bsz, num_tiles, ntok, width = x.shape
    nblk = (ntok + _BLKN - 1) // _BLKN
    emb2 = embedding.reshape(num_tiles * num_tiles, width)
    xt = jnp.transpose(x, (0, 2, 1, 3))

    import functools
    body = functools.partial(_body, num_tiles=num_tiles)

    def x_map(b, n, ar_ref, gate_ref):
        return (b, n, 0, 0)

    grid_spec = pltpu.PrefetchScalarGridSpec(
        num_scalar_prefetch=2,
        grid=(bsz, nblk),
        in_specs=[
            pl.BlockSpec((1, _BLKN, num_tiles, width), x_map),
            pl.BlockSpec(emb2.shape, lambda b, n, *_: (0, 0)),
        ],
        out_specs=pl.BlockSpec((1, _BLKN, num_tiles, width), x_map),
        scratch_shapes=[
            pltpu.VMEM((num_tiles, width), jnp.float32),
        ],
    )
    out_t = pl.pallas_call(
        body,
        grid_spec=grid_spec,
        out_shape=jax.ShapeDtypeStruct(xt.shape, x.dtype),
        compiler_params=pltpu.CompilerParams(
            dimension_semantics=("parallel", "arbitrary"),
            vmem_limit_bytes=117440512,
        ),
    )(ar, gate, xt, emb2)
    return jnp.transpose(out_t, (0, 2, 1, 3))
